# baseline (speedup 1.0000x reference)
import jax
import jax.numpy as jnp
from jax.experimental import pallas as pl
from jax.experimental.pallas import tpu as pltpu

_LANE = 128


def _softmax_dim0_block(x_ref, o_ref):
    e = jnp.exp(x_ref[...])
    part = jnp.sum(e, axis=0, keepdims=True)
    o_ref[...] = e * pl.reciprocal(part, approx=False)


def _pick_tile_d(N, D):
    target = 512
    for t in range(target, _LANE - 1, -_LANE):
        if D % t == 0:
            return t
    return D


def kernel(x):
    orig_shape = x.shape
    N = orig_shape[0]
    x2 = x.reshape(N, -1) if x.ndim != 2 else x
    D = x2.shape[1]

    tile_d = _pick_tile_d(N, D)
    cost = pl.CostEstimate(
        flops=2 * N * D,
        transcendentals=N * D,
        bytes_accessed=2 * N * D * x2.dtype.itemsize,
    )
    out = pl.pallas_call(
        _softmax_dim0_block,
        out_shape=jax.ShapeDtypeStruct((N, D), x2.dtype),
        grid=(D // tile_d,),
        in_specs=[pl.BlockSpec((N, tile_d), lambda j: (0, j))],
        out_specs=pl.BlockSpec((N, tile_d), lambda j: (0, j)),
        compiler_params=pltpu.CompilerParams(
            dimension_semantics=("parallel",),
            vmem_limit_bytes=64 * 1024 * 1024,
        ),
        cost_estimate=cost,
    )(x2)
    return out.reshape(orig_shape)
